# baseline (speedup 1.0000x reference)
import functools
import jax
import jax.numpy as jnp
from jax import lax
from jax.experimental import pallas as pl
from jax.experimental.pallas import tpu as pltpu
from jax.experimental.pallas import tpu_sc as plsc

N = 100000
N_IN = 512
N_OUT = 256
E_FF = 6400000
E_REC = 640000
THRESH = N - N_OUT

NW = 32
FF_PER = E_FF // NW
REC_PER = E_REC // NW
C_FF = 20000
C_REC = 2000
L = 16
CAP = 512
M_ALLOC = L * CAP + C_FF // L + 128
NB = 8
NS = 4
NR = 4
UNROLL = 5


@functools.partial(
    pl.kernel,
    mesh=plsc.VectorSubcoreMesh(core_axis_name="c", subcore_axis_name="s"),
    compiler_params=pltpu.CompilerParams(needs_layout_passes=False),
    out_type=jax.ShapeDtypeStruct((NW, N_OUT), jnp.float32),
    scratch_types=[
        pltpu.VMEM((C_FF,), jnp.int32),
        pltpu.VMEM((C_FF,), jnp.int32),
        pltpu.VMEM((M_ALLOC,), jnp.int32),
        pltpu.VMEM((L * N_OUT,), jnp.float32),
        pltpu.VMEM((NB * L,), jnp.int32),
        pltpu.VMEM((NB * L,), jnp.float32),
        pltpu.VMEM((NR * L,), jnp.int32),
        pltpu.VMEM((NB * L,), jnp.float32),
        pltpu.VMEM((N_OUT,), jnp.float32),
        pltpu.SemaphoreType.DMA,
        pltpu.SemaphoreType.DMA,
        pltpu.SemaphoreType.DMA,
        pltpu.SemaphoreType.DMA,
        pltpu.SemaphoreType.DMA,
        pltpu.SemaphoreType.DMA,
        pltpu.SemaphoreType.DMA,
        pltpu.SemaphoreType.DMA,
        pltpu.SemaphoreType.DMA,
        pltpu.SemaphoreType.DMA,
    ],
)
def _sc_partials(hprev_hbm, hist_hbm, ffsrc_hbm, ffdst_hbm, ffw_hbm,
                 recsrc_hbm, recdst_hbm, recdelay_hbm, recw_hbm,
                 out_hbm,
                 buf0_v, buf1_v, enc_v, acc_v, srcg_v, wg_v,
                 d16_v, h16_v, outbuf_v,
                 sem_s0, sem_s1,
                 sg0, sg1, sg2, sg3, sg4, sg5, sg6, sg7):
    wid = lax.axis_index("s") * 2 + lax.axis_index("c")
    iota = lax.iota(jnp.int32, L)
    lane_base = iota * CAP
    bufs = (buf0_v, buf1_v)
    ssems = (sem_s0, sem_s1)
    gsems = (sg0, sg1, sg2, sg3, sg4, sg5, sg6, sg7)

    def zacc(i, _):
        acc_v[pl.ds(i * L, L)] = jnp.zeros((L,), jnp.float32)
        return 0
    lax.fori_loop(0, (L * N_OUT) // L, zacc, 0)

    def stream_scan(dst_hbm, base, C_, nchunks):
        last = nchunks - 1

        def issue(c, b):
            pltpu.async_copy(dst_hbm.at[pl.ds(base + c * C_, C_)],
                             bufs[b].at[pl.ds(0, C_)], ssems[b])

        def wait(b):
            pltpu.make_async_copy(dst_hbm.at[pl.ds(0, C_)],
                                  bufs[b].at[pl.ds(0, C_)], ssems[b]).wait()

        def scan_buf(b, c, posv):
            posv = jnp.minimum(posv, lane_base + CAP)
            encbase0 = ((c * C_ + iota) << 8) - THRESH

            def vb(i, carry):
                posv, encbase = carry
                for u in range(UNROLL):
                    off = i * (L * UNROLL) + u * L
                    d = bufs[b][pl.ds(off, L)]
                    m = d >= THRESH
                    plsc.store_scatter(enc_v, [posv], d + encbase, mask=m)
                    posv = posv + m.astype(jnp.int32)
                    encbase = encbase + (L << 8)
                return (posv, encbase)

            posv, _ = lax.fori_loop(0, C_ // (L * UNROLL), vb,
                                    (posv, encbase0))
            return posv

        issue(0, 0)
        issue(1, 1)

        def pair_body(k, posv):
            c0 = 2 * k
            wait(0)
            posv = scan_buf(0, c0, posv)
            issue(jnp.minimum(c0 + 2, last), 0)
            wait(1)
            posv = scan_buf(1, c0 + 1, posv)
            issue(jnp.minimum(c0 + 3, last), 1)
            return posv

        posv = lax.fori_loop(0, nchunks // 2, pair_body, lane_base)
        if nchunks % 2:
            wait(0)
            posv = scan_buf(0, last, posv)
            wait(1)
        else:
            wait(0)
            wait(1)
        return jnp.minimum(posv - lane_base, CAP)

    def e_at(r):
        return plsc.load_gather(enc_v, [lane_base + jnp.minimum(r, CAP - 1)])

    def ff_groups(n_vec):
        rmax = jnp.max(n_vec)

        def issue_a(r, p):
            e = e_at(r)
            gi = jnp.where(r < n_vec, e >> 8, 0)
            gidx = wid * FF_PER + gi
            pltpu.async_copy(ffsrc_hbm.at[gidx],
                             srcg_v.at[pl.ds(p * L, L)], gsems[p])
            pltpu.async_copy(ffw_hbm.at[gidx],
                             wg_v.at[pl.ds(p * L, L)], gsems[p])

        def wait_a(p):
            pltpu.make_async_copy(ffsrc_hbm.at[pl.ds(0, L)],
                                  srcg_v.at[pl.ds(p * L, L)], gsems[p]).wait()
            pltpu.make_async_copy(ffw_hbm.at[pl.ds(0, L)],
                                  wg_v.at[pl.ds(p * L, L)], gsems[p]).wait()

        def issue_b(p):
            pltpu.async_copy(hprev_hbm.at[srcg_v[pl.ds(p * L, L)]],
                             h16_v.at[pl.ds(p * L, L)], gsems[p])

        def wait_b(p):
            pltpu.make_async_copy(hprev_hbm.at[pl.ds(0, L)],
                                  h16_v.at[pl.ds(p * L, L)], gsems[p]).wait()

        def compute(r, p):
            e = e_at(r)
            valid = r < n_vec
            dl = e & (N_OUT - 1)
            val = jnp.where(valid,
                            wg_v[pl.ds(p * L, L)] * h16_v[pl.ds(p * L, L)],
                            0.0)
            plsc.addupdate_scatter(acc_v, [(iota << 8) | dl], val)

        for p in range(NB):
            issue_a(jnp.int32(p), p)
        for q in range(NS):
            wait_a(q)
            issue_b(q)

        def k_body(k, _):
            for p in range(NB):
                r = k * NB + p
                q = p + NS
                if q < NB:
                    wait_a(q)
                    issue_b(q)
                wait_b(p)
                compute(r, p)
                issue_a(r + NB, p)
            for q in range(NS):
                wait_a(q)
                issue_b(q)
            return 0

        lax.fori_loop(0, (rmax + NB - 1) >> 3, k_body, 0)
        for q in range(NS):
            wait_b(q)
        for p in range(NS, NB):
            wait_a(p)

    def rec_groups(n_vec):
        rmax = jnp.max(n_vec)
        asems = (sg0, sg1, sg2, sg3)
        bsems = (sg4, sg5, sg6, sg7)

        def issue_a(r, p):
            e = e_at(r)
            gi = jnp.where(r < n_vec, e >> 8, 0)
            gidx = wid * REC_PER + gi
            pltpu.async_copy(recsrc_hbm.at[gidx],
                             srcg_v.at[pl.ds(p * L, L)], asems[p])
            pltpu.async_copy(recw_hbm.at[gidx],
                             wg_v.at[pl.ds(p * L, L)], asems[p])
            pltpu.async_copy(recdelay_hbm.at[gidx],
                             d16_v.at[pl.ds(p * L, L)], asems[p])

        def wait_a(p):
            pltpu.make_async_copy(recsrc_hbm.at[pl.ds(0, L)],
                                  srcg_v.at[pl.ds(p * L, L)], asems[p]).wait()
            pltpu.make_async_copy(recw_hbm.at[pl.ds(0, L)],
                                  wg_v.at[pl.ds(p * L, L)], asems[p]).wait()
            pltpu.make_async_copy(recdelay_hbm.at[pl.ds(0, L)],
                                  d16_v.at[pl.ds(p * L, L)], asems[p]).wait()

        def issue_b(p):
            flat = d16_v[pl.ds(p * L, L)] * N + srcg_v[pl.ds(p * L, L)]
            pltpu.async_copy(hist_hbm.at[flat],
                             h16_v.at[pl.ds(p * L, L)], bsems[p])

        def wait_b(p):
            pltpu.make_async_copy(hist_hbm.at[pl.ds(0, L)],
                                  h16_v.at[pl.ds(p * L, L)], bsems[p]).wait()

        def compute(r, p):
            e = e_at(r)
            valid = r < n_vec
            dl = e & (N_OUT - 1)
            val = jnp.where(valid,
                            wg_v[pl.ds(p * L, L)] * h16_v[pl.ds(p * L, L)],
                            0.0)
            plsc.addupdate_scatter(acc_v, [(iota << 8) | dl], val)

        for p in range(NR):
            issue_a(jnp.int32(p), p)
        wait_a(0)
        issue_b(0)

        def k_body(k, _):
            for p in range(NR):
                r = k * NR + p
                if p + 1 < NR:
                    wait_a(p + 1)
                    issue_b(p + 1)
                wait_b(p)
                compute(r, p)
                issue_a(r + NR, p)
            wait_a(0)
            issue_b(0)
            return 0

        lax.fori_loop(0, (rmax + NR - 1) >> 2, k_body, 0)
        wait_b(0)
        for p in range(1, NR):
            wait_a(p)

    n_ff = stream_scan(ffdst_hbm, wid * FF_PER, C_FF, FF_PER // C_FF)
    ff_groups(n_ff)
    n_rec = stream_scan(recdst_hbm, wid * REC_PER, C_REC, REC_PER // C_REC)
    rec_groups(n_rec)

    def red_body(j, _):
        def inner(l, s):
            return s + acc_v[pl.ds(l * N_OUT + j * L, L)]
        outbuf_v[pl.ds(j * L, L)] = lax.fori_loop(
            0, L, inner, jnp.zeros((L,), jnp.float32))
        return 0
    lax.fori_loop(0, N_OUT // L, red_body, 0)
    pltpu.sync_copy(outbuf_v, out_hbm.at[wid])


def _tc_combine(p_ref, b_ref, o_ref):
    o_ref[:, :] = jnp.tanh(b_ref[:, :] +
                           jnp.sum(p_ref[:, :], axis=0, keepdims=True))


def kernel(obs, h_state, hist, bias, ff_w, rec_w,
           ff_src, ff_dst, rec_src, rec_dst, rec_delay):
    h_prev = jnp.concatenate([obs, h_state[N_IN:]])
    hist_flat = hist.reshape(-1)
    partials = _sc_partials(h_prev, hist_flat, ff_src, ff_dst, ff_w,
                            rec_src, rec_dst, rec_delay, rec_w)
    bias_tail = bias[N - N_OUT:].reshape(1, N_OUT)
    out = pl.pallas_call(
        _tc_combine,
        out_shape=jax.ShapeDtypeStruct((1, N_OUT), jnp.float32),
    )(partials, bias_tail)
    return out.reshape(N_OUT)


# speedup vs baseline: 1.1909x; 1.1909x over previous
import functools
import numpy as np
import jax
import jax.numpy as jnp
from jax import lax
from jax.experimental import pallas as pl
from jax.experimental.pallas import tpu as pltpu
from jax.experimental.pallas import tpu_sc as plsc

N = 100000
N_IN = 512
N_OUT = 256
E_FF = 6400000
E_REC = 640000
THRESH = N - N_OUT

NW = 32
FF_PER = E_FF // NW
REC_PER = E_REC // NW
W_FF = FF_PER // 16
W_REC = REC_PER // 16
W_FF_P = 12800
W_REC_P = 1280
C_BM = 6400
L = 16
CAP = 512
WCAP = 128
M_ALLOC = L * CAP + 64
W_ALLOC = L * WCAP + C_BM // L + 64
NB = 8
NS = 4
NR = 4
RB = 1000


@functools.partial(
    pl.kernel,
    mesh=plsc.VectorSubcoreMesh(core_axis_name="c", subcore_axis_name="s"),
    compiler_params=pltpu.CompilerParams(needs_layout_passes=False),
    out_type=jax.ShapeDtypeStruct((NW, N_OUT), jnp.float32),
    scratch_types=[
        pltpu.VMEM((C_BM,), jnp.int32),
        pltpu.VMEM((C_BM,), jnp.int32),
        pltpu.VMEM((W_ALLOC,), jnp.int32),
        pltpu.VMEM((W_ALLOC,), jnp.int32),
        pltpu.VMEM((M_ALLOC,), jnp.int32),
        pltpu.VMEM((L * N_OUT,), jnp.float32),
        pltpu.VMEM((NB * L,), jnp.int32),
        pltpu.VMEM((NB * L,), jnp.float32),
        pltpu.VMEM((NB * L,), jnp.int32),
        pltpu.VMEM((NR * L,), jnp.int32),
        pltpu.VMEM((NB * L,), jnp.float32),
        pltpu.VMEM((N_OUT,), jnp.float32),
        pltpu.SemaphoreType.DMA,
        pltpu.SemaphoreType.DMA,
        pltpu.SemaphoreType.DMA,
        pltpu.SemaphoreType.DMA,
        pltpu.SemaphoreType.DMA,
        pltpu.SemaphoreType.DMA,
        pltpu.SemaphoreType.DMA,
        pltpu.SemaphoreType.DMA,
        pltpu.SemaphoreType.DMA,
        pltpu.SemaphoreType.DMA,
    ],
)
def _sc_partials(hprev_hbm, hist_hbm, ffbm_hbm, ffsrc_hbm, ffdst_hbm,
                 ffw_hbm, recbm_hbm, recsrc_hbm, recdst_hbm, recdelay_hbm,
                 recw_hbm, out_hbm,
                 buf0_v, buf1_v, wid_v, wval_v, enc_v, acc_v,
                 srcg_v, wg_v, dstg_v, d16_v, h16_v, outbuf_v,
                 sem_s0, sem_s1,
                 sg0, sg1, sg2, sg3, sg4, sg5, sg6, sg7):
    wid = lax.axis_index("s") * 2 + lax.axis_index("c")
    iota = lax.iota(jnp.int32, L)
    lane_base = iota * CAP
    wlane_base = iota * WCAP
    bufs = (buf0_v, buf1_v)
    ssems = (sem_s0, sem_s1)
    gsems = (sg0, sg1, sg2, sg3, sg4, sg5, sg6, sg7)

    def zacc(i, _):
        acc_v[pl.ds(i * L, L)] = jnp.zeros((L,), jnp.float32)
        return 0
    lax.fori_loop(0, (L * N_OUT) // L, zacc, 0)

    def word_scan(bm_hbm, base, C_, nchunks):
        last = nchunks - 1

        def issue(c, b):
            pltpu.async_copy(bm_hbm.at[pl.ds(base + c * C_, C_)],
                             bufs[b].at[pl.ds(0, C_)], ssems[b])

        def wait(b):
            pltpu.make_async_copy(bm_hbm.at[pl.ds(0, C_)],
                                  bufs[b].at[pl.ds(0, C_)], ssems[b]).wait()

        def scan_buf(b, c, posw):
            posw = jnp.minimum(posw, wlane_base + WCAP)
            widx0 = c * C_ + iota

            def vb(i, carry):
                posw, widx = carry
                for u in range(5):
                    off = i * (L * 5) + u * L
                    w = bufs[b][pl.ds(off, L)]
                    m = w != 0
                    plsc.store_scatter(wid_v, [posw], widx, mask=m)
                    plsc.store_scatter(wval_v, [posw], w, mask=m)
                    posw = posw + m.astype(jnp.int32)
                    widx = widx + L
                return (posw, widx)

            posw, _ = lax.fori_loop(0, C_ // (L * 5), vb, (posw, widx0))
            return posw

        issue(0, 0)
        issue(1, 1)

        def pair_body(k, posw):
            c0 = 2 * k
            wait(0)
            posw = scan_buf(0, c0, posw)
            issue(jnp.minimum(c0 + 2, last), 0)
            wait(1)
            posw = scan_buf(1, c0 + 1, posw)
            issue(jnp.minimum(c0 + 3, last), 1)
            return posw

        posw = lax.fori_loop(0, nchunks // 2, pair_body, wlane_base)
        if nchunks % 2:
            wait(0)
            posw = scan_buf(0, last, posw)
            wait(1)
        else:
            wait(0)
            wait(1)
        return jnp.minimum(posw - wlane_base, WCAP)

    def extract(nw):
        nwmax = jnp.max(nw)

        def l2_body(r, posv):
            rr = wlane_base + jnp.minimum(r, WCAP - 1)
            wv = plsc.load_gather(wval_v, [rr])
            wv = jnp.where(r < nw, wv, 0)
            wi = plsc.load_gather(wid_v, [rr]) << 4
            posv = jnp.minimum(posv, lane_base + CAP)
            for j in range(16):
                mj = ((wv >> j) & 1) == 1
                plsc.store_scatter(enc_v, [posv], wi | j, mask=mj)
                posv = posv + mj.astype(jnp.int32)
            return posv

        posv = lax.fori_loop(0, nwmax, l2_body, lane_base)
        return jnp.minimum(posv - lane_base, CAP)

    def e_at(r, n_vec):
        e = plsc.load_gather(enc_v, [lane_base + jnp.minimum(r, CAP - 1)])
        return jnp.where(r < n_vec, e, 0)

    def ff_groups(n_vec):
        rmax = jnp.max(n_vec)

        def issue_a(r, p):
            gidx = wid * FF_PER + e_at(r, n_vec)
            pltpu.async_copy(ffsrc_hbm.at[gidx],
                             srcg_v.at[pl.ds(p * L, L)], gsems[p])
            pltpu.async_copy(ffw_hbm.at[gidx],
                             wg_v.at[pl.ds(p * L, L)], gsems[p])
            pltpu.async_copy(ffdst_hbm.at[gidx],
                             dstg_v.at[pl.ds(p * L, L)], gsems[p])

        def wait_a(p):
            pltpu.make_async_copy(ffsrc_hbm.at[pl.ds(0, L)],
                                  srcg_v.at[pl.ds(p * L, L)], gsems[p]).wait()
            pltpu.make_async_copy(ffw_hbm.at[pl.ds(0, L)],
                                  wg_v.at[pl.ds(p * L, L)], gsems[p]).wait()
            pltpu.make_async_copy(ffdst_hbm.at[pl.ds(0, L)],
                                  dstg_v.at[pl.ds(p * L, L)], gsems[p]).wait()

        def issue_b(p):
            pltpu.async_copy(hprev_hbm.at[srcg_v[pl.ds(p * L, L)]],
                             h16_v.at[pl.ds(p * L, L)], gsems[p])

        def wait_b(p):
            pltpu.make_async_copy(hprev_hbm.at[pl.ds(0, L)],
                                  h16_v.at[pl.ds(p * L, L)], gsems[p]).wait()

        def compute(r, p):
            valid = r < n_vec
            dl = jnp.where(valid, dstg_v[pl.ds(p * L, L)] - THRESH, 0)
            val = jnp.where(valid,
                            wg_v[pl.ds(p * L, L)] * h16_v[pl.ds(p * L, L)],
                            0.0)
            plsc.addupdate_scatter(acc_v, [(iota << 8) | dl], val)

        for p in range(NB):
            issue_a(jnp.int32(p), p)
        for q in range(NS):
            wait_a(q)
            issue_b(q)

        def k_body(k, _):
            for p in range(NB):
                r = k * NB + p
                q = p + NS
                if q < NB:
                    wait_a(q)
                    issue_b(q)
                wait_b(p)
                compute(r, p)
                issue_a(r + NB, p)
            for q in range(NS):
                wait_a(q)
                issue_b(q)
            return 0

        lax.fori_loop(0, (rmax + NB - 1) >> 3, k_body, 0)
        for q in range(NS):
            wait_b(q)
        for p in range(NS, NB):
            wait_a(p)

    def rec_groups(n_vec):
        rmax = jnp.max(n_vec)
        asems = (sg0, sg1, sg2, sg3)
        bsems = (sg4, sg5, sg6, sg7)

        def issue_a(r, p):
            gidx = wid * REC_PER + e_at(r, n_vec)
            pltpu.async_copy(recsrc_hbm.at[gidx],
                             srcg_v.at[pl.ds(p * L, L)], asems[p])
            pltpu.async_copy(recw_hbm.at[gidx],
                             wg_v.at[pl.ds(p * L, L)], asems[p])
            pltpu.async_copy(recdelay_hbm.at[gidx],
                             d16_v.at[pl.ds(p * L, L)], asems[p])
            pltpu.async_copy(recdst_hbm.at[gidx],
                             dstg_v.at[pl.ds(p * L, L)], asems[p])

        def wait_a(p):
            pltpu.make_async_copy(recsrc_hbm.at[pl.ds(0, L)],
                                  srcg_v.at[pl.ds(p * L, L)], asems[p]).wait()
            pltpu.make_async_copy(recw_hbm.at[pl.ds(0, L)],
                                  wg_v.at[pl.ds(p * L, L)], asems[p]).wait()
            pltpu.make_async_copy(recdelay_hbm.at[pl.ds(0, L)],
                                  d16_v.at[pl.ds(p * L, L)], asems[p]).wait()
            pltpu.make_async_copy(recdst_hbm.at[pl.ds(0, L)],
                                  dstg_v.at[pl.ds(p * L, L)], asems[p]).wait()

        def issue_b(p):
            flat = d16_v[pl.ds(p * L, L)] * N + srcg_v[pl.ds(p * L, L)]
            pltpu.async_copy(hist_hbm.at[flat],
                             h16_v.at[pl.ds(p * L, L)], bsems[p])

        def wait_b(p):
            pltpu.make_async_copy(hist_hbm.at[pl.ds(0, L)],
                                  h16_v.at[pl.ds(p * L, L)], bsems[p]).wait()

        def compute(r, p):
            valid = r < n_vec
            dl = jnp.where(valid, dstg_v[pl.ds(p * L, L)] - THRESH, 0)
            val = jnp.where(valid,
                            wg_v[pl.ds(p * L, L)] * h16_v[pl.ds(p * L, L)],
                            0.0)
            plsc.addupdate_scatter(acc_v, [(iota << 8) | dl], val)

        for p in range(NR):
            issue_a(jnp.int32(p), p)
        wait_a(0)
        issue_b(0)

        def k_body(k, _):
            for p in range(NR):
                r = k * NR + p
                if p + 1 < NR:
                    wait_a(p + 1)
                    issue_b(p + 1)
                wait_b(p)
                compute(r, p)
                issue_a(r + NR, p)
            wait_a(0)
            issue_b(0)
            return 0

        lax.fori_loop(0, (rmax + NR - 1) >> 2, k_body, 0)
        wait_b(0)
        for p in range(1, NR):
            wait_a(p)

    nw_ff = word_scan(ffbm_hbm, wid * W_FF_P, C_BM, W_FF_P // C_BM)
    n_ff = extract(nw_ff)
    ff_groups(n_ff)
    nw_rec = word_scan(recbm_hbm, wid * W_REC_P, W_REC_P, 1)
    n_rec = extract(nw_rec)
    rec_groups(n_rec)

    def red_body(j, _):
        def inner(l, s):
            return s + acc_v[pl.ds(l * N_OUT + j * L, L)]
        outbuf_v[pl.ds(j * L, L)] = lax.fori_loop(
            0, L, inner, jnp.zeros((L,), jnp.float32))
        return 0
    lax.fori_loop(0, N_OUT // L, red_body, 0)
    pltpu.sync_copy(outbuf_v, out_hbm.at[wid])


def _tc_bitmap_body(dst_ref, pw_ref, o_ref):
    m = (dst_ref[:, :] >= THRESH).astype(jnp.float32)
    o_ref[:, :] = jnp.dot(m, pw_ref[:, :]).astype(jnp.int32)


def _tc_bitmap(dst, pw):
    nrows = dst.shape[0]
    return pl.pallas_call(
        _tc_bitmap_body,
        grid=(nrows // RB,),
        in_specs=[
            pl.BlockSpec((RB, 128), lambda i: (i, 0)),
            pl.BlockSpec((128, 8), lambda i: (0, 0)),
        ],
        out_specs=pl.BlockSpec((RB, 8), lambda i: (i, 0)),
        out_shape=jax.ShapeDtypeStruct((nrows, 8), jnp.int32),
    )(dst, pw)


def _tc_combine(p_ref, b_ref, o_ref):
    o_ref[:, :] = jnp.tanh(b_ref[:, :] +
                           jnp.sum(p_ref[:, :], axis=0, keepdims=True))


_PW = np.zeros((128, 8), np.float32)
for _j in range(128):
    _PW[_j, _j // 16] = float(2 ** (_j % 16))


def kernel(obs, h_state, hist, bias, ff_w, rec_w,
           ff_src, ff_dst, rec_src, rec_dst, rec_delay):
    h_prev = jnp.concatenate([obs, h_state[N_IN:]])
    hist_flat = hist.reshape(-1)
    pw = jnp.asarray(_PW)
    ff_bm = _tc_bitmap(ff_dst.reshape(E_FF // 128, 128), pw)
    ff_bm = jnp.pad(ff_bm.reshape(NW, W_FF),
                    ((0, 0), (0, W_FF_P - W_FF))).reshape(-1)
    rec_bm = _tc_bitmap(rec_dst.reshape(E_REC // 128, 128), pw)
    rec_bm = jnp.pad(rec_bm.reshape(NW, W_REC),
                     ((0, 0), (0, W_REC_P - W_REC))).reshape(-1)
    partials = _sc_partials(h_prev, hist_flat, ff_bm, ff_src, ff_dst, ff_w,
                            rec_bm, rec_src, rec_dst, rec_delay, rec_w)
    bias_tail = bias[N - N_OUT:].reshape(1, N_OUT)
    out = pl.pallas_call(
        _tc_combine,
        out_shape=jax.ShapeDtypeStruct((1, N_OUT), jnp.float32),
    )(partials, bias_tail)
    return out.reshape(N_OUT)
